# Initial kernel scaffold; baseline (speedup 1.0000x reference)
import jax
import jax.numpy as jnp
from jax.experimental import pallas as pl


def kernel(images, world_points, world_points_conf, extrinsic, intrinsic):
    raise NotImplementedError("write your pallas kernel here")


# speedup vs baseline: 29.9312x; 29.9312x over previous
import dataclasses

import jax
import jax.numpy as jnp
from jax import lax
from jax.experimental import pallas as pl
from jax.experimental.pallas import tpu as pltpu
from jax.experimental.pallas import tpu_sc as plsc

S, H, W = 64, 224, 224
HW = H * W
N = S * HW
NC, NS, L = 2, 16, 16
NW = NC * NS
FPW = S // NW
CHUNK = 3584
NCHUNK = HW // CHUNK

_INF = float("inf")


def _b16(x):
    return x.astype(jnp.bfloat16).astype(jnp.float32)


def _proj_body(m_ref, px_ref, py_ref, pz_ref, u_ref, v_ref, z_ref):
    px = _b16(px_ref[0])
    py = _b16(py_ref[0])
    pz = _b16(pz_ref[0])

    def m(k):
        return m_ref[0, 0, k]

    c0 = m(0) * px + m(1) * py + m(2) * pz + m(3)
    c1 = m(4) * px + m(5) * py + m(6) * pz + m(7)
    c2 = m(8) * px + m(9) * py + m(10) * pz + m(11)
    z = c2
    c0b = _b16(c0)
    c1b = _b16(c1)
    c2b = _b16(c2)
    a0 = m(12) * c0b + m(13) * c1b + m(14) * c2b
    a1 = m(15) * c0b + m(16) * c1b + m(17) * c2b
    a2 = m(18) * c0b + m(19) * c1b + m(20) * c2b
    den = a2 + 1e-7
    u = a0 / den
    v = a1 / den
    valid = (z > 0) & (u >= 0) & (u < float(W)) & (v >= 0) & (v < float(H))
    uc = jnp.clip(u, 0.0, float(W - 1))
    vc = jnp.clip(v, 0.0, float(H - 1))
    u_ref[0] = jnp.where(valid, uc, -1.0 - uc)
    v_ref[0] = vc
    z_ref[0] = z


def _project(world_points, extrinsic, intrinsic):
    pts = world_points.reshape(S, HW, 3)
    px = pts[:, :, 0].reshape(S, 392, 128)
    py = pts[:, :, 1].reshape(S, 392, 128)
    pz = pts[:, :, 2].reshape(S, 392, 128)
    Eb = lax.reduce_precision(extrinsic, 8, 7)
    Kb = lax.reduce_precision(intrinsic, 8, 7)
    M = jnp.concatenate([Eb.reshape(S, 12), Kb.reshape(S, 9),
                         jnp.zeros((S, 3), jnp.float32)], axis=1)
    M = M.reshape(S, 1, 24)
    bs = pl.BlockSpec((1, 392, 128), lambda s: (s, 0, 0))
    u, v, z = pl.pallas_call(
        _proj_body,
        grid=(S,),
        in_specs=[
            pl.BlockSpec((1, 1, 24), lambda s: (s, 0, 0),
                         memory_space=pltpu.SMEM),
            bs, bs, bs,
        ],
        out_specs=[bs, bs, bs],
        out_shape=[jax.ShapeDtypeStruct((S, 392, 128), jnp.float32)] * 3,
    )(M, px, py, pz)
    return u.reshape(N), v.reshape(N), z.reshape(N)


def _minimum_scatter(depth_v, idx, zval):
    d0 = plsc.load_gather(depth_v, [idx])
    m0 = jnp.minimum(d0, zval)
    plsc.store_scatter(depth_v, [idx], m0)
    g0 = plsc.load_gather(depth_v, [idx])

    def cond(carry):
        return jnp.max(carry[1], axis=0) > 0

    def body(carry):
        m, lost_i = carry
        plsc.store_scatter(depth_v, [idx], m, mask=lost_i > 0)
        g = plsc.load_gather(depth_v, [idx])
        return (m, (g > m).astype(jnp.int32))

    lax.while_loop(cond, body, (m0, (g0 > m0).astype(jnp.int32)))


def _decode(u_enc):
    vm = u_enc >= 0.0
    u = jnp.where(vm, u_enc, -1.0 - u_enc)
    return u, vm


def _splat_body(uenc_hbm, v_hbm, z_hbm, conf_hbm, img_hbm,
                depth_out, conf_out, rgb_out,
                depth_v, acc_v, su, sv, sz, sval):
    wid = lax.axis_index("s") * NC + lax.axis_index("c")

    for fi in range(FPW):
        f = wid * FPW + fi
        base = f * HW

        @pl.loop(0, HW, step=L)
        def _(i):
            depth_v[pl.ds(i, L)] = jnp.full((L,), _INF, jnp.float32)

        @pl.loop(0, NCHUNK)
        def _(c):
            off = base + c * CHUNK
            pltpu.sync_copy(uenc_hbm.at[pl.ds(off, CHUNK)], su)
            pltpu.sync_copy(v_hbm.at[pl.ds(off, CHUNK)], sv)
            pltpu.sync_copy(z_hbm.at[pl.ds(off, CHUNK)], sz)

            @pl.loop(0, CHUNK, step=L)
            def _(j):
                u, _vm = _decode(su[pl.ds(j, L)])
                vv = sv[pl.ds(j, L)]
                zz = sz[pl.ds(j, L)]
                idx = vv.astype(jnp.int32) * W + u.astype(jnp.int32)
                _minimum_scatter(depth_v, idx, zz)

        pltpu.sync_copy(depth_v, depth_out.at[pl.ds(base, HW)])

        for ch in range(4):
            if ch == 0:
                src, src_base = conf_hbm, base
                dst, dst_base = conf_out, base
            else:
                src, src_base = img_hbm, (f * 3 + (ch - 1)) * HW
                dst, dst_base = rgb_out, (f * 3 + (ch - 1)) * HW

            @pl.loop(0, HW, step=L)
            def _(i):
                acc_v[pl.ds(i, L)] = jnp.zeros((L,), jnp.float32)

            @pl.loop(0, NCHUNK)
            def _(c):
                off = c * CHUNK
                pltpu.sync_copy(uenc_hbm.at[pl.ds(base + off, CHUNK)], su)
                pltpu.sync_copy(v_hbm.at[pl.ds(base + off, CHUNK)], sv)
                pltpu.sync_copy(z_hbm.at[pl.ds(base + off, CHUNK)], sz)
                pltpu.sync_copy(src.at[pl.ds(src_base + off, CHUNK)], sval)

                @pl.loop(0, CHUNK, step=L)
                def _(j):
                    u, vm = _decode(su[pl.ds(j, L)])
                    vv = sv[pl.ds(j, L)]
                    zz = sz[pl.ds(j, L)]
                    val = sval[pl.ds(j, L)]
                    u0 = u.astype(jnp.int32)
                    v0 = vv.astype(jnp.int32)
                    idx00 = v0 * W + u0
                    d = plsc.load_gather(depth_v, [idx00])
                    hit = (zz == d) & vm
                    hf = jnp.where(hit, val, 0.0)
                    du = u - u0.astype(jnp.float32)
                    dv = vv - v0.astype(jnp.float32)
                    p = (1.0 - du) * hf
                    q = du * hf
                    s_u = jnp.minimum(u0 + 1, W - 1) - u0
                    s_v = (jnp.minimum(v0 + 1, H - 1) - v0) * W
                    idx01 = idx00 + s_v
                    idx11 = idx01 + s_u
                    plsc.addupdate_scatter(acc_v, [idx00], p * (1.0 - dv))
                    plsc.addupdate_scatter(acc_v, [idx01], p * dv)
                    plsc.addupdate_scatter(acc_v, [idx00 + s_u], q * (1.0 - dv))
                    plsc.addupdate_scatter(acc_v, [idx11], q * dv)

            pltpu.sync_copy(acc_v, dst.at[pl.ds(dst_base, HW)])


def _splat_sc(u_enc, v, z, conf, images_flat):
    mesh = plsc.VectorSubcoreMesh(core_axis_name="c", subcore_axis_name="s")
    cp = pltpu.CompilerParams()
    if "needs_layout_passes" in pltpu.CompilerParams.__dataclass_fields__:
        cp = dataclasses.replace(cp, needs_layout_passes=False)
    kfn = pl.kernel(
        _splat_body,
        out_type=[
            jax.ShapeDtypeStruct((N,), jnp.float32),
            jax.ShapeDtypeStruct((N,), jnp.float32),
            jax.ShapeDtypeStruct((3 * N,), jnp.float32),
        ],
        mesh=mesh,
        scratch_types=[
            pltpu.VMEM((HW,), jnp.float32),
            pltpu.VMEM((HW,), jnp.float32),
            pltpu.VMEM((CHUNK,), jnp.float32),
            pltpu.VMEM((CHUNK,), jnp.float32),
            pltpu.VMEM((CHUNK,), jnp.float32),
            pltpu.VMEM((CHUNK,), jnp.float32),
        ],
        compiler_params=cp,
    )
    return kfn(u_enc, v, z, conf, images_flat)


def _final_body(d_ref, rgb_ref, rgbo_ref, mask_ref):
    d = d_ref[0]
    r = rgb_ref[0, 0]
    g = rgb_ref[0, 1]
    b = rgb_ref[0, 2]
    over = (r > 1.0) | (g > 1.0) | (b > 1.0)
    rgbo_ref[0, 0] = jnp.clip(r, 0.0, 1.0)
    rgbo_ref[0, 1] = jnp.clip(g, 0.0, 1.0)
    rgbo_ref[0, 2] = jnp.clip(b, 0.0, 1.0)
    mask_ref[0] = ((d < _INF) & (~over)).astype(jnp.float32)


def _finalize(depth_raw, rgb_raw):
    d = depth_raw.reshape(S, 392, 128)
    rgb = rgb_raw.reshape(S, 3, 392, 128)
    bsd = pl.BlockSpec((1, 392, 128), lambda s: (s, 0, 0))
    bsr = pl.BlockSpec((1, 3, 392, 128), lambda s: (s, 0, 0, 0))
    rgbo, mask = pl.pallas_call(
        _final_body,
        grid=(S,),
        in_specs=[bsd, bsr],
        out_specs=[bsr, bsd],
        out_shape=[jax.ShapeDtypeStruct((S, 3, 392, 128), jnp.float32),
                   jax.ShapeDtypeStruct((S, 392, 128), jnp.float32)],
    )(d, rgb)
    return rgbo, mask


def _kernel_ab(images, world_points, world_points_conf, extrinsic, intrinsic):
    f32 = jnp.float32
    u_enc, v, z = _project(world_points, extrinsic, intrinsic)
    vmb = u_enc >= 0.0
    u = jnp.where(vmb, u_enc, -1.0 - u_enc)
    vm = vmb.astype(f32)
    conf = world_points_conf.reshape(N)
    rgb = jnp.transpose(images, (0, 2, 3, 1)).reshape(N, 3)
    b_idx = jnp.repeat(jnp.arange(S, dtype=jnp.int32), HW)
    pix_idx = b_idx * HW + v.astype(jnp.int32) * W + u.astype(jnp.int32)
    u0 = u.astype(jnp.int32)
    u1 = jnp.minimum(u0 + 1, W - 1)
    v0 = v.astype(jnp.int32)
    v1 = jnp.minimum(v0 + 1, H - 1)
    du = u - u0.astype(f32)
    dv = v - v0.astype(f32)
    w00 = (1 - du) * (1 - dv) * vm
    w01 = (1 - du) * dv * vm
    w10 = du * (1 - dv) * vm
    w11 = du * dv * vm
    pix00 = b_idx * HW + v0 * W + u0
    pix01 = b_idx * HW + v1 * W + u0
    pix10 = b_idx * HW + v0 * W + u1
    pix11 = b_idx * HW + v1 * W + u1
    depth_buf = jnp.full((N,), jnp.inf, dtype=f32).at[pix_idx].min(z)
    hit = (z == depth_buf[pix_idx]) & vmb
    hitf = hit.astype(f32)
    rgb_buf = jnp.zeros((N, 3), dtype=f32)
    conf_buf = jnp.zeros((N,), dtype=f32)
    for w, pix in ((w00, pix00), (w01, pix01), (w10, pix10), (w11, pix11)):
        wh = w * hitf
        rgb_buf = rgb_buf.at[pix].add(wh[:, None] * rgb)
        conf_buf = conf_buf.at[pix].add(wh * conf)
    over_exposed = (rgb_buf > 1.0).any(axis=1)
    rgb_buf = jnp.clip(rgb_buf, 0.0, 1.0)
    depth_map = jnp.transpose(depth_buf.reshape(S, H, W, 1), (0, 3, 1, 2))
    rgb_map = jnp.transpose(rgb_buf.reshape(S, H, W, 3), (0, 3, 1, 2))
    conf_map = jnp.transpose(conf_buf.reshape(S, H, W, 1), (0, 3, 1, 2))
    mask_map = ((depth_buf < jnp.inf) & (~over_exposed)).reshape(S, 1, H, W).astype(f32)
    return (rgb_map, depth_map, conf_map, mask_map)


def kernel(images, world_points, world_points_conf, extrinsic, intrinsic):
    u_enc, v, z = _project(world_points, extrinsic, intrinsic)
    conf = world_points_conf.reshape(N)
    images_flat = images.reshape(3 * N)
    depth_raw, conf_raw, rgb_raw = _splat_sc(u_enc, v, z, conf, images_flat)
    rgbo, mask = _finalize(depth_raw, rgb_raw)
    rgb_map = rgbo.reshape(S, 3, H, W)
    depth_map = depth_raw.reshape(S, 1, H, W)
    conf_map = conf_raw.reshape(S, 1, H, W)
    mask_map = mask.reshape(S, 1, H, W)
    return (rgb_map, depth_map, conf_map, mask_map)


# speedup vs baseline: 35.4513x; 1.1844x over previous
import dataclasses

import jax
import jax.numpy as jnp
from jax import lax
from jax.experimental import pallas as pl
from jax.experimental.pallas import tpu as pltpu
from jax.experimental.pallas import tpu_sc as plsc

S, H, W = 64, 224, 224
HW = H * W
N = S * HW
NC, NS, L = 2, 16, 16
NW = NC * NS
FPW = S // NW
CHUNK = 3136
NCHUNK = HW // CHUNK

_INF = float("inf")


def _b16(x):
    return x.astype(jnp.bfloat16).astype(jnp.float32)


def _proj_body(m_ref, px_ref, py_ref, pz_ref, u_ref, v_ref, z_ref):
    px = _b16(px_ref[0])
    py = _b16(py_ref[0])
    pz = _b16(pz_ref[0])

    def m(k):
        return m_ref[0, 0, k]

    c0 = m(0) * px + m(1) * py + m(2) * pz + m(3)
    c1 = m(4) * px + m(5) * py + m(6) * pz + m(7)
    c2 = m(8) * px + m(9) * py + m(10) * pz + m(11)
    z = c2
    c0b = _b16(c0)
    c1b = _b16(c1)
    c2b = _b16(c2)
    a0 = m(12) * c0b + m(13) * c1b + m(14) * c2b
    a1 = m(15) * c0b + m(16) * c1b + m(17) * c2b
    a2 = m(18) * c0b + m(19) * c1b + m(20) * c2b
    den = a2 + 1e-7
    u = a0 / den
    v = a1 / den
    valid = (z > 0) & (u >= 0) & (u < float(W)) & (v >= 0) & (v < float(H))
    uc = jnp.clip(u, 0.0, float(W - 1))
    vc = jnp.clip(v, 0.0, float(H - 1))
    u_ref[0] = jnp.where(valid, uc, -1.0 - uc)
    v_ref[0] = vc
    z_ref[0] = z


def _project(world_points, extrinsic, intrinsic):
    pts = world_points.reshape(S, HW, 3)
    px = pts[:, :, 0].reshape(S, 392, 128)
    py = pts[:, :, 1].reshape(S, 392, 128)
    pz = pts[:, :, 2].reshape(S, 392, 128)
    Eb = lax.reduce_precision(extrinsic, 8, 7)
    Kb = lax.reduce_precision(intrinsic, 8, 7)
    M = jnp.concatenate([Eb.reshape(S, 12), Kb.reshape(S, 9),
                         jnp.zeros((S, 3), jnp.float32)], axis=1)
    M = M.reshape(S, 1, 24)
    bs = pl.BlockSpec((1, 392, 128), lambda s: (s, 0, 0))
    u, v, z = pl.pallas_call(
        _proj_body,
        grid=(S,),
        in_specs=[
            pl.BlockSpec((1, 1, 24), lambda s: (s, 0, 0),
                         memory_space=pltpu.SMEM),
            bs, bs, bs,
        ],
        out_specs=[bs, bs, bs],
        out_shape=[jax.ShapeDtypeStruct((S, 392, 128), jnp.float32)] * 3,
    )(M, px, py, pz)
    return u.reshape(N), v.reshape(N), z.reshape(N)


def _minimum_scatter(depth_v, idx, zval):
    d0 = plsc.load_gather(depth_v, [idx])
    m0 = jnp.minimum(d0, zval)
    plsc.store_scatter(depth_v, [idx], m0)
    g0 = plsc.load_gather(depth_v, [idx])
    nlost = plsc.all_reduce_population_count(g0 > m0)

    @pl.when(nlost[0] > 0)
    def _():
        def cond(carry):
            return jnp.max(carry, axis=0) > 0

        def body(carry):
            plsc.store_scatter(depth_v, [idx], m0, mask=carry > 0)
            g = plsc.load_gather(depth_v, [idx])
            return (g > m0).astype(jnp.int32)

        lax.while_loop(cond, body, (g0 > m0).astype(jnp.int32))


def _decode(u_enc):
    vm = u_enc >= 0.0
    u = jnp.where(vm, u_enc, -1.0 - u_enc)
    return u, vm


def _phase_a_vregs(depth_v, bufs):
    su, sv, sz = bufs[0], bufs[1], bufs[2]

    @pl.loop(0, CHUNK, step=L)
    def _(j):
        u, _vm = _decode(su[pl.ds(j, L)])
        vv = sv[pl.ds(j, L)]
        zz = sz[pl.ds(j, L)]
        idx = vv.astype(jnp.int32) * W + u.astype(jnp.int32)
        _minimum_scatter(depth_v, idx, zz)


def _phase_b_vregs(depth_v, acc_v, bufs):
    su, sv, sz, sval = bufs

    @pl.loop(0, CHUNK, step=L)
    def _(j):
        u, vm = _decode(su[pl.ds(j, L)])
        vv = sv[pl.ds(j, L)]
        zz = sz[pl.ds(j, L)]
        val = sval[pl.ds(j, L)]
        u0 = u.astype(jnp.int32)
        v0 = vv.astype(jnp.int32)
        idx00 = v0 * W + u0
        d = plsc.load_gather(depth_v, [idx00])
        hit = (zz == d) & vm
        hf = jnp.where(hit, val, 0.0)
        du = u - u0.astype(jnp.float32)
        dv = vv - v0.astype(jnp.float32)
        p = (1.0 - du) * hf
        q = du * hf
        s_u = jnp.minimum(u0 + 1, W - 1) - u0
        s_v = (jnp.minimum(v0 + 1, H - 1) - v0) * W
        idx01 = idx00 + s_v
        idx11 = idx01 + s_u
        plsc.addupdate_scatter(acc_v, [idx00], p * (1.0 - dv))
        plsc.addupdate_scatter(acc_v, [idx01], p * dv)
        plsc.addupdate_scatter(acc_v, [idx00 + s_u], q * (1.0 - dv))
        plsc.addupdate_scatter(acc_v, [idx11], q * dv)


def _pipelined_pass(srcs, src_bases, bufs0, bufs1, sem0, sem1, process):
    n = len(srcs)

    def start(bufs, c, sem):
        for k in range(n):
            pltpu.async_copy(
                srcs[k].at[pl.ds(src_bases[k] + c * CHUNK, CHUNK)],
                bufs[k], sem)

    def drain(bufs, sem):
        for k in range(n):
            pltpu.make_async_copy(
                srcs[k].at[pl.ds(src_bases[k], CHUNK)], bufs[k], sem).wait()

    start(bufs0, 0, sem0)

    @pl.loop(0, NCHUNK, step=2)
    def _(c):
        start(bufs1, c + 1, sem1)
        drain(bufs0, sem0)
        process(bufs0)

        @pl.when(c + 2 < NCHUNK)
        def _():
            start(bufs0, c + 2, sem0)

        drain(bufs1, sem1)
        process(bufs1)


def _splat_body(uenc_hbm, v_hbm, z_hbm, conf_hbm, img_hbm,
                depth_out, conf_out, rgb_out,
                depth_v, acc_v,
                su0, sv0, sz0, sval0, su1, sv1, sz1, sval1, sem0, sem1):
    wid = lax.axis_index("s") * NC + lax.axis_index("c")

    for fi in range(FPW):
        f = wid * FPW + fi
        base = f * HW

        @pl.loop(0, HW, step=L)
        def _(i):
            depth_v[pl.ds(i, L)] = jnp.full((L,), _INF, jnp.float32)

        _pipelined_pass(
            [uenc_hbm, v_hbm, z_hbm], [base, base, base],
            [su0, sv0, sz0], [su1, sv1, sz1], sem0, sem1,
            lambda bufs: _phase_a_vregs(depth_v, bufs))

        pltpu.sync_copy(depth_v, depth_out.at[pl.ds(base, HW)])

        for ch in range(4):
            if ch == 0:
                src, src_base = conf_hbm, base
                dst, dst_base = conf_out, base
            else:
                src, src_base = img_hbm, (f * 3 + (ch - 1)) * HW
                dst, dst_base = rgb_out, (f * 3 + (ch - 1)) * HW

            @pl.loop(0, HW, step=L)
            def _(i):
                acc_v[pl.ds(i, L)] = jnp.zeros((L,), jnp.float32)

            _pipelined_pass(
                [uenc_hbm, v_hbm, z_hbm, src], [base, base, base, src_base],
                [su0, sv0, sz0, sval0], [su1, sv1, sz1, sval1], sem0, sem1,
                lambda bufs: _phase_b_vregs(depth_v, acc_v, bufs))

            pltpu.sync_copy(acc_v, dst.at[pl.ds(dst_base, HW)])


def _splat_sc(u_enc, v, z, conf, images_flat):
    mesh = plsc.VectorSubcoreMesh(core_axis_name="c", subcore_axis_name="s")
    cp = pltpu.CompilerParams()
    if "needs_layout_passes" in pltpu.CompilerParams.__dataclass_fields__:
        cp = dataclasses.replace(cp, needs_layout_passes=False)
    kfn = pl.kernel(
        _splat_body,
        out_type=[
            jax.ShapeDtypeStruct((N,), jnp.float32),
            jax.ShapeDtypeStruct((N,), jnp.float32),
            jax.ShapeDtypeStruct((3 * N,), jnp.float32),
        ],
        mesh=mesh,
        scratch_types=(
            [pltpu.VMEM((HW,), jnp.float32)] * 2
            + [pltpu.VMEM((CHUNK,), jnp.float32)] * 8
            + [pltpu.SemaphoreType.DMA] * 2
        ),
        compiler_params=cp,
    )
    return kfn(u_enc, v, z, conf, images_flat)


def _final_body(d_ref, rgb_ref, rgbo_ref, mask_ref):
    d = d_ref[0]
    r = rgb_ref[0, 0]
    g = rgb_ref[0, 1]
    b = rgb_ref[0, 2]
    over = (r > 1.0) | (g > 1.0) | (b > 1.0)
    rgbo_ref[0, 0] = jnp.clip(r, 0.0, 1.0)
    rgbo_ref[0, 1] = jnp.clip(g, 0.0, 1.0)
    rgbo_ref[0, 2] = jnp.clip(b, 0.0, 1.0)
    mask_ref[0] = ((d < _INF) & (~over)).astype(jnp.float32)


def _finalize(depth_raw, rgb_raw):
    d = depth_raw.reshape(S, 392, 128)
    rgb = rgb_raw.reshape(S, 3, 392, 128)
    bsd = pl.BlockSpec((1, 392, 128), lambda s: (s, 0, 0))
    bsr = pl.BlockSpec((1, 3, 392, 128), lambda s: (s, 0, 0, 0))
    rgbo, mask = pl.pallas_call(
        _final_body,
        grid=(S,),
        in_specs=[bsd, bsr],
        out_specs=[bsr, bsd],
        out_shape=[jax.ShapeDtypeStruct((S, 3, 392, 128), jnp.float32),
                   jax.ShapeDtypeStruct((S, 392, 128), jnp.float32)],
    )(d, rgb)
    return rgbo, mask


def _kernel_ab(images, world_points, world_points_conf, extrinsic, intrinsic):
    f32 = jnp.float32
    u_enc, v, z = _project(world_points, extrinsic, intrinsic)
    vmb = u_enc >= 0.0
    u = jnp.where(vmb, u_enc, -1.0 - u_enc)
    vm = vmb.astype(f32)
    conf = world_points_conf.reshape(N)
    rgb = jnp.transpose(images, (0, 2, 3, 1)).reshape(N, 3)
    b_idx = jnp.repeat(jnp.arange(S, dtype=jnp.int32), HW)
    pix_idx = b_idx * HW + v.astype(jnp.int32) * W + u.astype(jnp.int32)
    u0 = u.astype(jnp.int32)
    u1 = jnp.minimum(u0 + 1, W - 1)
    v0 = v.astype(jnp.int32)
    v1 = jnp.minimum(v0 + 1, H - 1)
    du = u - u0.astype(f32)
    dv = v - v0.astype(f32)
    w00 = (1 - du) * (1 - dv) * vm
    w01 = (1 - du) * dv * vm
    w10 = du * (1 - dv) * vm
    w11 = du * dv * vm
    pix00 = b_idx * HW + v0 * W + u0
    pix01 = b_idx * HW + v1 * W + u0
    pix10 = b_idx * HW + v0 * W + u1
    pix11 = b_idx * HW + v1 * W + u1
    depth_buf = jnp.full((N,), jnp.inf, dtype=f32).at[pix_idx].min(z)
    hit = (z == depth_buf[pix_idx]) & vmb
    hitf = hit.astype(f32)
    rgb_buf = jnp.zeros((N, 3), dtype=f32)
    conf_buf = jnp.zeros((N,), dtype=f32)
    for w, pix in ((w00, pix00), (w01, pix01), (w10, pix10), (w11, pix11)):
        wh = w * hitf
        rgb_buf = rgb_buf.at[pix].add(wh[:, None] * rgb)
        conf_buf = conf_buf.at[pix].add(wh * conf)
    over_exposed = (rgb_buf > 1.0).any(axis=1)
    rgb_buf = jnp.clip(rgb_buf, 0.0, 1.0)
    depth_map = jnp.transpose(depth_buf.reshape(S, H, W, 1), (0, 3, 1, 2))
    rgb_map = jnp.transpose(rgb_buf.reshape(S, H, W, 3), (0, 3, 1, 2))
    conf_map = jnp.transpose(conf_buf.reshape(S, H, W, 1), (0, 3, 1, 2))
    mask_map = ((depth_buf < jnp.inf) & (~over_exposed)).reshape(S, 1, H, W).astype(f32)
    return (rgb_map, depth_map, conf_map, mask_map)


def kernel(images, world_points, world_points_conf, extrinsic, intrinsic):
    u_enc, v, z = _project(world_points, extrinsic, intrinsic)
    conf = world_points_conf.reshape(N)
    images_flat = images.reshape(3 * N)
    depth_raw, conf_raw, rgb_raw = _splat_sc(u_enc, v, z, conf, images_flat)
    rgbo, mask = _finalize(depth_raw, rgb_raw)
    rgb_map = rgbo.reshape(S, 3, H, W)
    depth_map = depth_raw.reshape(S, 1, H, W)
    conf_map = conf_raw.reshape(S, 1, H, W)
    mask_map = mask.reshape(S, 1, H, W)
    return (rgb_map, depth_map, conf_map, mask_map)


# speedup vs baseline: 35.7097x; 1.0073x over previous
import dataclasses

import jax
import jax.numpy as jnp
from jax import lax
from jax.experimental import pallas as pl
from jax.experimental.pallas import tpu as pltpu
from jax.experimental.pallas import tpu_sc as plsc

S, H, W = 64, 224, 224
HW = H * W
N = S * HW
NC, NS, L = 2, 16, 16
NW = NC * NS
FPW = S // NW
CHUNK = 3136
NCHUNK = HW // CHUNK

_INF = float("inf")


def _b16(x):
    return x.astype(jnp.bfloat16).astype(jnp.float32)


def _proj_body(m_ref, px_ref, py_ref, pz_ref, u_ref, v_ref, z_ref):
    px = _b16(px_ref[0])
    py = _b16(py_ref[0])
    pz = _b16(pz_ref[0])

    def m(k):
        return m_ref[0, 0, k]

    c0 = m(0) * px + m(1) * py + m(2) * pz + m(3)
    c1 = m(4) * px + m(5) * py + m(6) * pz + m(7)
    c2 = m(8) * px + m(9) * py + m(10) * pz + m(11)
    z = c2
    c0b = _b16(c0)
    c1b = _b16(c1)
    c2b = _b16(c2)
    a0 = m(12) * c0b + m(13) * c1b + m(14) * c2b
    a1 = m(15) * c0b + m(16) * c1b + m(17) * c2b
    a2 = m(18) * c0b + m(19) * c1b + m(20) * c2b
    den = a2 + 1e-7
    u = a0 / den
    v = a1 / den
    valid = (z > 0) & (u >= 0) & (u < float(W)) & (v >= 0) & (v < float(H))
    uc = jnp.clip(u, 0.0, float(W - 1))
    vc = jnp.clip(v, 0.0, float(H - 1))
    u_ref[0] = jnp.where(valid, uc, -1.0 - uc)
    v_ref[0] = vc
    z_ref[0] = z


def _project(world_points, extrinsic, intrinsic):
    pts = world_points.reshape(S, HW, 3)
    px = pts[:, :, 0].reshape(S, 392, 128)
    py = pts[:, :, 1].reshape(S, 392, 128)
    pz = pts[:, :, 2].reshape(S, 392, 128)
    Eb = lax.reduce_precision(extrinsic, 8, 7)
    Kb = lax.reduce_precision(intrinsic, 8, 7)
    M = jnp.concatenate([Eb.reshape(S, 12), Kb.reshape(S, 9),
                         jnp.zeros((S, 3), jnp.float32)], axis=1)
    M = M.reshape(S, 1, 24)
    bs = pl.BlockSpec((1, 392, 128), lambda s: (s, 0, 0))
    u, v, z = pl.pallas_call(
        _proj_body,
        grid=(S,),
        in_specs=[
            pl.BlockSpec((1, 1, 24), lambda s: (s, 0, 0),
                         memory_space=pltpu.SMEM),
            bs, bs, bs,
        ],
        out_specs=[bs, bs, bs],
        out_shape=[jax.ShapeDtypeStruct((S, 392, 128), jnp.float32)] * 3,
    )(M, px, py, pz)
    return u.reshape(N), v.reshape(N), z.reshape(N)


def _minimum_scatter(depth_v, idx, zval):
    d0 = plsc.load_gather(depth_v, [idx])
    m0 = jnp.minimum(d0, zval)
    plsc.store_scatter(depth_v, [idx], m0)
    g0 = plsc.load_gather(depth_v, [idx])
    nlost = plsc.all_reduce_population_count(g0 > m0)

    @pl.when(nlost[0] > 0)
    def _():
        def cond(carry):
            return jnp.max(carry, axis=0) > 0

        def body(carry):
            plsc.store_scatter(depth_v, [idx], m0, mask=carry > 0)
            g = plsc.load_gather(depth_v, [idx])
            return (g > m0).astype(jnp.int32)

        lax.while_loop(cond, body, (g0 > m0).astype(jnp.int32))


def _decode(u_enc):
    vm = u_enc >= 0.0
    u = jnp.where(vm, u_enc, -1.0 - u_enc)
    return u, vm


def _phase_a_vregs(depth_v, bufs):
    su, sv, sz = bufs[0], bufs[1], bufs[2]

    def one(j):
        u, _vm = _decode(su[pl.ds(j, L)])
        vv = sv[pl.ds(j, L)]
        zz = sz[pl.ds(j, L)]
        idx = vv.astype(jnp.int32) * W + u.astype(jnp.int32)
        _minimum_scatter(depth_v, idx, zz)

    @pl.loop(0, CHUNK, step=L)
    def _(j):
        one(j)


def _phase_b_vregs(depth_v, acc_v, bufs):
    su, sv, sz, sval = bufs

    def one(j):
        u, vm = _decode(su[pl.ds(j, L)])
        vv = sv[pl.ds(j, L)]
        zz = sz[pl.ds(j, L)]
        val = sval[pl.ds(j, L)]
        u0 = u.astype(jnp.int32)
        v0 = vv.astype(jnp.int32)
        idx00 = v0 * W + u0
        d = plsc.load_gather(depth_v, [idx00])
        hit = (zz == d) & vm
        hf = jnp.where(hit, val, 0.0)
        du = u - u0.astype(jnp.float32)
        dv = vv - v0.astype(jnp.float32)
        p = (1.0 - du) * hf
        q = du * hf
        s_u = jnp.minimum(u0 + 1, W - 1) - u0
        s_v = (jnp.minimum(v0 + 1, H - 1) - v0) * W
        idx01 = idx00 + s_v
        idx11 = idx01 + s_u
        plsc.addupdate_scatter(acc_v, [idx00], p * (1.0 - dv))
        plsc.addupdate_scatter(acc_v, [idx01], p * dv)
        plsc.addupdate_scatter(acc_v, [idx00 + s_u], q * (1.0 - dv))
        plsc.addupdate_scatter(acc_v, [idx11], q * dv)

    @pl.loop(0, CHUNK, step=2 * L)
    def _(j):
        one(j)
        one(j + L)


def _pipelined_pass(srcs, src_bases, bufs0, bufs1, sem0, sem1, process):
    n = len(srcs)

    def start(bufs, c, sem):
        for k in range(n):
            pltpu.async_copy(
                srcs[k].at[pl.ds(src_bases[k] + c * CHUNK, CHUNK)],
                bufs[k], sem)

    def drain(bufs, sem):
        for k in range(n):
            pltpu.make_async_copy(
                srcs[k].at[pl.ds(src_bases[k], CHUNK)], bufs[k], sem).wait()

    start(bufs0, 0, sem0)

    @pl.loop(0, NCHUNK, step=2)
    def _(c):
        start(bufs1, c + 1, sem1)
        drain(bufs0, sem0)
        process(bufs0)

        @pl.when(c + 2 < NCHUNK)
        def _():
            start(bufs0, c + 2, sem0)

        drain(bufs1, sem1)
        process(bufs1)


def _splat_body(uenc_hbm, v_hbm, z_hbm, conf_hbm, img_hbm,
                depth_out, conf_out, rgb_out,
                depth_v, acc_v,
                su0, sv0, sz0, sval0, su1, sv1, sz1, sval1, sem0, sem1):
    wid = lax.axis_index("s") * NC + lax.axis_index("c")

    for fi in range(FPW):
        f = wid * FPW + fi
        base = f * HW

        @pl.loop(0, HW, step=L)
        def _(i):
            depth_v[pl.ds(i, L)] = jnp.full((L,), _INF, jnp.float32)

        _pipelined_pass(
            [uenc_hbm, v_hbm, z_hbm], [base, base, base],
            [su0, sv0, sz0], [su1, sv1, sz1], sem0, sem1,
            lambda bufs: _phase_a_vregs(depth_v, bufs))

        pltpu.sync_copy(depth_v, depth_out.at[pl.ds(base, HW)])

        for ch in range(4):
            if ch == 0:
                src, src_base = conf_hbm, base
                dst, dst_base = conf_out, base
            else:
                src, src_base = img_hbm, (f * 3 + (ch - 1)) * HW
                dst, dst_base = rgb_out, (f * 3 + (ch - 1)) * HW

            @pl.loop(0, HW, step=L)
            def _(i):
                acc_v[pl.ds(i, L)] = jnp.zeros((L,), jnp.float32)

            _pipelined_pass(
                [uenc_hbm, v_hbm, z_hbm, src], [base, base, base, src_base],
                [su0, sv0, sz0, sval0], [su1, sv1, sz1, sval1], sem0, sem1,
                lambda bufs: _phase_b_vregs(depth_v, acc_v, bufs))

            pltpu.sync_copy(acc_v, dst.at[pl.ds(dst_base, HW)])


def _splat_sc(u_enc, v, z, conf, images_flat):
    mesh = plsc.VectorSubcoreMesh(core_axis_name="c", subcore_axis_name="s")
    cp = pltpu.CompilerParams()
    if "needs_layout_passes" in pltpu.CompilerParams.__dataclass_fields__:
        cp = dataclasses.replace(cp, needs_layout_passes=False)
    kfn = pl.kernel(
        _splat_body,
        out_type=[
            jax.ShapeDtypeStruct((N,), jnp.float32),
            jax.ShapeDtypeStruct((N,), jnp.float32),
            jax.ShapeDtypeStruct((3 * N,), jnp.float32),
        ],
        mesh=mesh,
        scratch_types=(
            [pltpu.VMEM((HW,), jnp.float32)] * 2
            + [pltpu.VMEM((CHUNK,), jnp.float32)] * 8
            + [pltpu.SemaphoreType.DMA] * 2
        ),
        compiler_params=cp,
    )
    return kfn(u_enc, v, z, conf, images_flat)


def _final_body(d_ref, rgb_ref, rgbo_ref, mask_ref):
    d = d_ref[0]
    r = rgb_ref[0, 0]
    g = rgb_ref[0, 1]
    b = rgb_ref[0, 2]
    over = (r > 1.0) | (g > 1.0) | (b > 1.0)
    rgbo_ref[0, 0] = jnp.clip(r, 0.0, 1.0)
    rgbo_ref[0, 1] = jnp.clip(g, 0.0, 1.0)
    rgbo_ref[0, 2] = jnp.clip(b, 0.0, 1.0)
    mask_ref[0] = ((d < _INF) & (~over)).astype(jnp.float32)


def _finalize(depth_raw, rgb_raw):
    d = depth_raw.reshape(S, 392, 128)
    rgb = rgb_raw.reshape(S, 3, 392, 128)
    bsd = pl.BlockSpec((1, 392, 128), lambda s: (s, 0, 0))
    bsr = pl.BlockSpec((1, 3, 392, 128), lambda s: (s, 0, 0, 0))
    rgbo, mask = pl.pallas_call(
        _final_body,
        grid=(S,),
        in_specs=[bsd, bsr],
        out_specs=[bsr, bsd],
        out_shape=[jax.ShapeDtypeStruct((S, 3, 392, 128), jnp.float32),
                   jax.ShapeDtypeStruct((S, 392, 128), jnp.float32)],
    )(d, rgb)
    return rgbo, mask


def kernel(images, world_points, world_points_conf, extrinsic, intrinsic):
    u_enc, v, z = _project(world_points, extrinsic, intrinsic)
    conf = world_points_conf.reshape(N)
    images_flat = images.reshape(3 * N)
    depth_raw, conf_raw, rgb_raw = _splat_sc(u_enc, v, z, conf, images_flat)
    rgbo, mask = _finalize(depth_raw, rgb_raw)
    rgb_map = rgbo.reshape(S, 3, H, W)
    depth_map = depth_raw.reshape(S, 1, H, W)
    conf_map = conf_raw.reshape(S, 1, H, W)
    mask_map = mask.reshape(S, 1, H, W)
    return (rgb_map, depth_map, conf_map, mask_map)
